# baseline (speedup 1.0000x reference)
import functools

import jax
import jax.numpy as jnp
from jax import lax
from jax.experimental import pallas as pl
from jax.experimental.pallas import tpu as pltpu
from jax.experimental.pallas import tpu_sc as plsc

B = 16384
E = 16
NC = 2
NS = 16
NW = NC * NS
CHUNK = B // NW
LANES = 16


def _sc_gather_partials(idx_u, idx_n, user_emb, news_emb, user_b, news_b):
    mesh = plsc.VectorSubcoreMesh(core_axis_name="c", subcore_axis_name="s")

    @functools.partial(
        pl.kernel,
        out_type=(
            jax.ShapeDtypeStruct((NW, LANES), jnp.float32),
            jax.ShapeDtypeStruct((B,), jnp.float32),
        ),
        mesh=mesh,
        compiler_params=pltpu.CompilerParams(use_tc_tiling_on_sc=False),
        scratch_types=[
            pltpu.VMEM((CHUNK,), jnp.int32),
            pltpu.VMEM((CHUNK,), jnp.int32),
            pltpu.VMEM((CHUNK, E), jnp.float32),
            pltpu.VMEM((CHUNK, E), jnp.float32),
            pltpu.VMEM((CHUNK,), jnp.float32),
            pltpu.VMEM((CHUNK,), jnp.float32),
            pltpu.VMEM((LANES,), jnp.float32),
            pltpu.VMEM((CHUNK,), jnp.float32),
            pltpu.SemaphoreType.DMA,
            pltpu.SemaphoreType.DMA,
            pltpu.SemaphoreType.DMA,
            pltpu.SemaphoreType.DMA,
        ],
    )
    def k(idxu_hbm, idxn_hbm, ue_hbm, ne_hbm, ubt_hbm, nbt_hbm,
          part_hbm, bsum_hbm,
          idxu_v, idxn_v, urows_v, nrows_v, ub_v, nb_v, acc_v, bs_v,
          sem0, sem1, sem2, sem3):
        wid = lax.axis_index("s") * NC + lax.axis_index("c")
        base = wid * CHUNK
        pltpu.sync_copy(idxu_hbm.at[pl.ds(base, CHUNK)], idxu_v)
        pltpu.sync_copy(idxn_hbm.at[pl.ds(base, CHUNK)], idxn_v)
        c0 = pltpu.async_copy(ue_hbm.at[idxu_v], urows_v, sem0)
        c1 = pltpu.async_copy(ne_hbm.at[idxn_v], nrows_v, sem1)
        c2 = pltpu.async_copy(ubt_hbm.at[idxu_v], ub_v, sem2)
        c3 = pltpu.async_copy(nbt_hbm.at[idxn_v], nb_v, sem3)
        c0.wait()
        c1.wait()

        def body(i, acc):
            return acc + urows_v[i, :] * nrows_v[i, :]

        acc = lax.fori_loop(0, CHUNK, body, jnp.zeros((LANES,), jnp.float32))
        acc_v[...] = acc
        pltpu.sync_copy(acc_v, part_hbm.at[wid])

        c2.wait()
        c3.wait()

        @pl.loop(0, CHUNK, step=LANES)
        def _(j):
            bs_v[pl.ds(j, LANES)] = ub_v[pl.ds(j, LANES)] + nb_v[pl.ds(j, LANES)]

        pltpu.sync_copy(bs_v, bsum_hbm.at[pl.ds(base, CHUNK)])

    return k(idx_u, idx_n, user_emb, news_emb, user_b, news_b)


def _tc_finish(partials, bias_sum):
    def body(p_ref, b_ref, o_ref):
        dot = jnp.sum(p_ref[...])
        o_ref[...] = jax.nn.sigmoid(b_ref[...] + dot)

    return pl.pallas_call(
        body,
        out_shape=jax.ShapeDtypeStruct((128, 128), jnp.float32),
    )(partials, bias_sum.reshape(128, 128))


def kernel(inputs, user_embedding, user_bias, news_embedding, news_bias):
    idx_u = inputs[:, 0]
    idx_n = inputs[:, 1]
    partials, bias_sum = _sc_gather_partials(
        idx_u, idx_n, user_embedding, news_embedding,
        user_bias.reshape(-1), news_bias.reshape(-1))
    out = _tc_finish(partials, bias_sum)
    return out.reshape(B, 1)
